# baseline (speedup 1.0000x reference)
import functools

import jax
import jax.numpy as jnp
from jax import lax
from jax.experimental import pallas as pl
from jax.experimental.pallas import tpu as pltpu
from jax.experimental.pallas import tpu_sc as plsc

N = 10000
E = 320000
D = 128

NC = 2
NS = 16
NW = NC * NS
N_PAD = 10240
EPW = 10560
E_PAD = NW * EPW
RPT = N_PAD // NS
ZR = 16


def _chunk_edges(d):
    return 80 if d == 128 else 64


@functools.lru_cache(maxsize=None)
def _make_segsum(d):
    mesh = plsc.VectorSubcoreMesh(core_axis_name="c", subcore_axis_name="s")
    k = _chunk_edges(d)
    nchunk = EPW // k
    halves = 2 if d == 128 else 1
    span = nchunk // halves

    @functools.partial(
        pl.kernel,
        mesh=mesh,
        compiler_params=pltpu.CompilerParams(use_tc_tiling_on_sc=False),
        out_type=jax.ShapeDtypeStruct((NC * N_PAD, d), jnp.float32),
        scratch_types=[
            pltpu.VMEM((span, k), jnp.int32),
            pltpu.VMEM((span, k), jnp.int32),
            pltpu.VMEM((k, d), jnp.float32),
            pltpu.VMEM((k, d), jnp.float32),
            pltpu.VMEM((k, d), jnp.float32),
            pltpu.VMEM((ZR, d), jnp.float32),
            pltpu.VMEM_SHARED((N_PAD, d), jnp.float32),
            pltpu.SemaphoreType.DMA,
            pltpu.SemaphoreType.DMA,
            pltpu.SemaphoreType.DMA,
            pltpu.SemaphoreType.DMA,
        ],
    )
    def segsum(x_hbm, src_hbm, dst_hbm, out_hbm, src_v, dst_v, buf0, buf1,
               buf2, zero_v, acc_sh, gsem0, gsem1, gsem2, isem):
        bufs = (buf0, buf1, buf2)
        gsems = (gsem0, gsem1, gsem2)
        c = lax.axis_index("c")
        s = lax.axis_index("s")
        gw = c * NS + s

        def load_slab(h):
            pltpu.async_copy(src_hbm.at[gw, pl.ds(h * span, span)], src_v, isem)
            pltpu.async_copy(dst_hbm.at[gw, pl.ds(h * span, span)], dst_v, isem)

        def wait_slab():
            pltpu.make_async_copy(src_hbm.at[gw, pl.ds(0, span)], src_v, isem).wait()
            pltpu.make_async_copy(dst_hbm.at[gw, pl.ds(0, span)], dst_v, isem).wait()

        load_slab(0)

        def zrow(i, carry):
            def zcol(j, carry2):
                zero_v[i, pl.ds(j * 16, 16)] = jnp.zeros((16,), jnp.float32)
                return carry2
            return lax.fori_loop(0, d // 16, zcol, carry)
        lax.fori_loop(0, ZR, zrow, 0)

        r0 = s * RPT
        def zacc(i, carry):
            pltpu.sync_copy(zero_v, acc_sh.at[pl.ds(r0 + i * ZR, ZR)])
            return carry
        lax.fori_loop(0, RPT // ZR, zacc, 0)
        wait_slab()
        plsc.subcore_barrier()

        def gather_start(j, q):
            pltpu.async_copy(x_hbm.at[src_v.at[j]], bufs[q], gsems[q])

        def gather_wait(j, q):
            pltpu.make_async_copy(x_hbm.at[src_v.at[j]], bufs[q], gsems[q]).wait()

        def scatter_add(j, q):
            pltpu.sync_copy(bufs[q], acc_sh.at[dst_v.at[j]], add=True)

        def run_span():
            gather_start(0, 0)
            gather_start(1, 1)

            def body(t, carry):
                j0 = 3 * t
                for q in range(3):
                    j = j0 + q
                    gather_wait(j, q)

                    @pl.when(j + 2 < span)
                    def _():
                        gather_start(j + 2, (q + 2) % 3)
                    scatter_add(j, q)
                return carry
            lax.fori_loop(0, span // 3, body, 0)

        assert span % 3 == 0 or halves == 1
        if halves == 1:
            if span % 3 == 0:
                run_span()
            else:
                gather_start(0, 0)

                def body2(i, carry):
                    j0 = 2 * i
                    gather_wait(j0, 0)
                    gather_start(j0 + 1, 1)
                    scatter_add(j0, 0)

                    @pl.when(j0 + 2 < span)
                    def _():
                        gather_start(j0 + 2, 0)
                    gather_wait(j0 + 1, 1)
                    scatter_add(j0 + 1, 1)
                    return carry
                lax.fori_loop(0, span // 2, body2, 0)
        else:
            run_span()
            load_slab(1)
            wait_slab()
            run_span()
        plsc.subcore_barrier()

        pltpu.sync_copy(acc_sh.at[pl.ds(r0, RPT)],
                        out_hbm.at[pl.ds(c * N_PAD + r0, RPT)])

    return segsum


def _bn(h, g, b):
    m = jnp.mean(h, axis=0, keepdims=True)
    v = jnp.mean(h * h, axis=0, keepdims=True) - m * m
    return (h - m) * lax.rsqrt(v + 1e-5) * g + b


def _mlp1_body(x, p, W1a, b1a, W1b, b1b, W1c, b1c, g1, be1, out):
    a = x[...] + p[0:N] + p[N_PAD:N_PAD + N]
    h = jnp.maximum(jnp.dot(a, W1a[...], preferred_element_type=jnp.float32) + b1a[...], 0.0)
    h = jnp.maximum(jnp.dot(h, W1b[...], preferred_element_type=jnp.float32) + b1b[...], 0.0)
    h = jnp.dot(h, W1c[...], preferred_element_type=jnp.float32) + b1c[...]
    h = jnp.maximum(h, 0.0)
    out[...] = _bn(h, g1[...], be1[...])


def _mlp2_body(x, p, W2a, b2a, W2b, b2b, W2c, b2c, g2, be2, out):
    a = x[...] + p[0:N] + p[N_PAD:N_PAD + N]
    h = jnp.maximum(jnp.dot(a, W2a[...], preferred_element_type=jnp.float32) + b2a[...], 0.0)
    h = jnp.maximum(jnp.dot(h, W2b[...], preferred_element_type=jnp.float32) + b2b[...], 0.0)
    h = jnp.dot(h, W2c[...], preferred_element_type=jnp.float32) + b2c[...]
    h = jnp.maximum(h, 0.0)
    out[...] = _bn(h, g2[...], be2[...])


def _mlp3_body(x, p, W3a, b3a, W3b, b3b, g3, be3, Wf1, bf1, Wf2, bf2, out):
    a = x[...] + p[0:N] + p[N_PAD:N_PAD + N]
    h = jnp.maximum(jnp.dot(a, W3a[...], preferred_element_type=jnp.float32) + b3a[...], 0.0)
    h = jnp.dot(h, W3b[...], preferred_element_type=jnp.float32) + b3b[...]
    h = jnp.maximum(h, 0.0)
    h = _bn(h, g3[...], be3[...])
    h = jnp.maximum(jnp.dot(h, Wf1[...], preferred_element_type=jnp.float32) + bf1[...], 0.0)
    h = jnp.dot(h, Wf2[...], preferred_element_type=jnp.float32) + bf2[...]
    out[...] = jnp.tanh(h)


def _tc_call(body, n_out):
    return pl.pallas_call(
        body,
        out_shape=jax.ShapeDtypeStruct((N, n_out), jnp.float32),
    )


def _r(v):
    return v.reshape(1, -1)


def kernel(x, edge_index, batch,
           W1a, b1a, W1b, b1b, W1c, b1c, g1, be1,
           W2a, b2a, W2b, b2b, W2c, b2c, g2, be2,
           W3a, b3a, W3b, b3b, g3, be3,
           Wf1, bf1, Wf2, bf2):
    src = edge_index[0]
    dst = edge_index[1]
    padn = E_PAD - E
    pad_iota = lax.iota(jnp.int32, padn)
    srcp = jnp.concatenate([src, pad_iota % N])
    dstp = jnp.concatenate([dst, N + pad_iota % (N_PAD - N)])

    def shaped(a, d):
        k = _chunk_edges(d)
        return a.reshape(NW, EPW // k, k)

    p1 = _make_segsum(128)(x, shaped(srcp, 128), shaped(dstp, 128))
    h1 = _tc_call(_mlp1_body, 128)(
        x, p1, W1a, _r(b1a), W1b, _r(b1b), W1c, _r(b1c), _r(g1), _r(be1))
    p2 = _make_segsum(128)(h1, shaped(srcp, 128), shaped(dstp, 128))
    h2 = _tc_call(_mlp2_body, 32)(
        h1, p2, W2a, _r(b2a), W2b, _r(b2b), W2c, _r(b2c), _r(g2), _r(be2))
    p3 = _make_segsum(32)(h2, shaped(srcp, 32), shaped(dstp, 32))
    out = _tc_call(_mlp3_body, 10)(
        h2, p3, W3a, _r(b3a), W3b, _r(b3b), _r(g3), _r(be3),
        Wf1, _r(bf1), Wf2, _r(bf2))
    return out


# speedup vs baseline: 1.0315x; 1.0315x over previous
import functools

import jax
import jax.numpy as jnp
from jax import lax
from jax.experimental import pallas as pl
from jax.experimental.pallas import tpu as pltpu
from jax.experimental.pallas import tpu_sc as plsc

N = 10000
E = 320000
D = 128

NC = 2
NS = 16
NW = NC * NS
N_PAD = 10240
EPW = 10560
E_PAD = NW * EPW
RPT = N_PAD // NS
ZR = 16


def _chunk_edges(d):
    return 80 if d == 128 else 120


@functools.lru_cache(maxsize=None)
def _make_segsum(d):
    mesh = plsc.VectorSubcoreMesh(core_axis_name="c", subcore_axis_name="s")
    k = _chunk_edges(d)
    nchunk = EPW // k
    halves = 2 if d == 128 else 1
    span = nchunk // halves

    @functools.partial(
        pl.kernel,
        mesh=mesh,
        compiler_params=pltpu.CompilerParams(use_tc_tiling_on_sc=False),
        out_type=jax.ShapeDtypeStruct((NC * N_PAD, d), jnp.float32),
        scratch_types=[
            pltpu.VMEM((span, k), jnp.int32),
            pltpu.VMEM((span, k), jnp.int32),
            pltpu.VMEM((k, d), jnp.float32),
            pltpu.VMEM((k, d), jnp.float32),
            pltpu.VMEM((k, d), jnp.float32),
            pltpu.VMEM((ZR, d), jnp.float32),
            pltpu.VMEM_SHARED((N_PAD, d), jnp.float32),
            pltpu.SemaphoreType.DMA,
            pltpu.SemaphoreType.DMA,
            pltpu.SemaphoreType.DMA,
            pltpu.SemaphoreType.DMA,
        ],
    )
    def segsum(x_hbm, src_hbm, dst_hbm, out_hbm, src_v, dst_v, buf0, buf1,
               buf2, zero_v, acc_sh, gsem0, gsem1, gsem2, isem):
        bufs = (buf0, buf1, buf2)
        gsems = (gsem0, gsem1, gsem2)
        c = lax.axis_index("c")
        s = lax.axis_index("s")
        gw = c * NS + s

        def load_slab(h):
            pltpu.async_copy(src_hbm.at[gw, pl.ds(h * span, span)], src_v, isem)
            pltpu.async_copy(dst_hbm.at[gw, pl.ds(h * span, span)], dst_v, isem)

        def wait_slab():
            pltpu.make_async_copy(src_hbm.at[gw, pl.ds(0, span)], src_v, isem).wait()
            pltpu.make_async_copy(dst_hbm.at[gw, pl.ds(0, span)], dst_v, isem).wait()

        load_slab(0)

        def zrow(i, carry):
            def zcol(j, carry2):
                zero_v[i, pl.ds(j * 16, 16)] = jnp.zeros((16,), jnp.float32)
                return carry2
            return lax.fori_loop(0, d // 16, zcol, carry)
        lax.fori_loop(0, ZR, zrow, 0)

        r0 = s * RPT
        def zacc(i, carry):
            pltpu.sync_copy(zero_v, acc_sh.at[pl.ds(r0 + i * ZR, ZR)])
            return carry
        lax.fori_loop(0, RPT // ZR, zacc, 0)
        wait_slab()
        plsc.subcore_barrier()

        def gather_start(j, q):
            pltpu.async_copy(x_hbm.at[src_v.at[j]], bufs[q], gsems[q])

        def gather_wait(j, q):
            pltpu.make_async_copy(x_hbm.at[src_v.at[j]], bufs[q], gsems[q]).wait()

        def scatter_add(j, q):
            pltpu.sync_copy(bufs[q], acc_sh.at[dst_v.at[j]], add=True)

        def run_span():
            gather_start(0, 0)
            gather_start(1, 1)

            def body(t, carry):
                j0 = 3 * t
                for q in range(3):
                    j = j0 + q
                    gather_wait(j, q)

                    @pl.when(j + 2 < span)
                    def _():
                        gather_start(j + 2, (q + 2) % 3)
                    scatter_add(j, q)
                return carry
            lax.fori_loop(0, span // 3, body, 0)

        assert span % 3 == 0 or halves == 1
        if halves == 1:
            if span % 3 == 0:
                run_span()
            else:
                gather_start(0, 0)

                def body2(i, carry):
                    j0 = 2 * i
                    gather_wait(j0, 0)
                    gather_start(j0 + 1, 1)
                    scatter_add(j0, 0)

                    @pl.when(j0 + 2 < span)
                    def _():
                        gather_start(j0 + 2, 0)
                    gather_wait(j0 + 1, 1)
                    scatter_add(j0 + 1, 1)
                    return carry
                lax.fori_loop(0, span // 2, body2, 0)
        else:
            run_span()
            load_slab(1)
            wait_slab()
            run_span()
        plsc.subcore_barrier()

        pltpu.sync_copy(acc_sh.at[pl.ds(r0, RPT)],
                        out_hbm.at[pl.ds(c * N_PAD + r0, RPT)])

    return segsum


def _bn(h, g, b):
    m = jnp.mean(h, axis=0, keepdims=True)
    v = jnp.mean(h * h, axis=0, keepdims=True) - m * m
    return (h - m) * lax.rsqrt(v + 1e-5) * g + b


def _mlp1_body(x, p, W1a, b1a, W1b, b1b, W1c, b1c, g1, be1, out):
    a = x[...] + p[0:N] + p[N_PAD:N_PAD + N]
    h = jnp.maximum(jnp.dot(a, W1a[...], preferred_element_type=jnp.float32) + b1a[...], 0.0)
    h = jnp.maximum(jnp.dot(h, W1b[...], preferred_element_type=jnp.float32) + b1b[...], 0.0)
    h = jnp.dot(h, W1c[...], preferred_element_type=jnp.float32) + b1c[...]
    h = jnp.maximum(h, 0.0)
    out[...] = _bn(h, g1[...], be1[...])


def _mlp2_body(x, p, W2a, b2a, W2b, b2b, W2c, b2c, g2, be2, out):
    a = x[...] + p[0:N] + p[N_PAD:N_PAD + N]
    h = jnp.maximum(jnp.dot(a, W2a[...], preferred_element_type=jnp.float32) + b2a[...], 0.0)
    h = jnp.maximum(jnp.dot(h, W2b[...], preferred_element_type=jnp.float32) + b2b[...], 0.0)
    h = jnp.dot(h, W2c[...], preferred_element_type=jnp.float32) + b2c[...]
    h = jnp.maximum(h, 0.0)
    out[...] = _bn(h, g2[...], be2[...])


def _mlp3_body(x, p, W3a, b3a, W3b, b3b, g3, be3, Wf1, bf1, Wf2, bf2, out):
    a = x[...] + p[0:N] + p[N_PAD:N_PAD + N]
    h = jnp.maximum(jnp.dot(a, W3a[...], preferred_element_type=jnp.float32) + b3a[...], 0.0)
    h = jnp.dot(h, W3b[...], preferred_element_type=jnp.float32) + b3b[...]
    h = jnp.maximum(h, 0.0)
    h = _bn(h, g3[...], be3[...])
    h = jnp.maximum(jnp.dot(h, Wf1[...], preferred_element_type=jnp.float32) + bf1[...], 0.0)
    h = jnp.dot(h, Wf2[...], preferred_element_type=jnp.float32) + bf2[...]
    out[...] = jnp.tanh(h)


def _tc_call(body, n_out):
    return pl.pallas_call(
        body,
        out_shape=jax.ShapeDtypeStruct((N, n_out), jnp.float32),
    )


def _r(v):
    return v.reshape(1, -1)


def kernel(x, edge_index, batch,
           W1a, b1a, W1b, b1b, W1c, b1c, g1, be1,
           W2a, b2a, W2b, b2b, W2c, b2c, g2, be2,
           W3a, b3a, W3b, b3b, g3, be3,
           Wf1, bf1, Wf2, bf2):
    src = edge_index[0]
    dst = edge_index[1]
    padn = E_PAD - E
    pad_iota = lax.iota(jnp.int32, padn)
    srcp = jnp.concatenate([src, pad_iota % N])
    dstp = jnp.concatenate([dst, N + pad_iota % (N_PAD - N)])

    def shaped(a, d):
        k = _chunk_edges(d)
        return a.reshape(NW, EPW // k, k)

    p1 = _make_segsum(128)(x, shaped(srcp, 128), shaped(dstp, 128))
    h1 = _tc_call(_mlp1_body, 128)(
        x, p1, W1a, _r(b1a), W1b, _r(b1b), W1c, _r(b1c), _r(g1), _r(be1))
    p2 = _make_segsum(128)(h1, shaped(srcp, 128), shaped(dstp, 128))
    h2 = _tc_call(_mlp2_body, 32)(
        h1, p2, W2a, _r(b2a), W2b, _r(b2b), W2c, _r(b2c), _r(g2), _r(be2))
    p3 = _make_segsum(32)(h2, shaped(srcp, 32), shaped(dstp, 32))
    out = _tc_call(_mlp3_body, 10)(
        h2, p3, W3a, _r(b3a), W3b, _r(b3b), _r(g3), _r(be3),
        Wf1, _r(bf1), Wf2, _r(bf2))
    return out
